# baseline (speedup 1.0000x reference)
import functools

import jax
import jax.numpy as jnp
from jax import lax
from jax.experimental import pallas as pl
from jax.experimental.pallas import tpu as pltpu
from jax.experimental.pallas import tpu_sc as plsc

N = 10000
E = 320000
NT = 32
R = 320
NPAD = NT * R
SCAN = 3200
NCH = E // SCAN
CAP = 324608
GCH = 128
IG = 16
RT = R + 1
CB = SCAN + 16

_mesh = plsc.VectorSubcoreMesh(core_axis_name="c", subcore_axis_name="s")
_sc_params = pltpu.CompilerParams(needs_layout_passes=False)


def _tile_id():
    return lax.axis_index("c") * 16 + lax.axis_index("s")


def _bin_body(src_hbm, dst_hbm, bin_pk, counts_hbm, deg_hbm,
              sbuf, dbuf, cbuf, deg_acc, cvec, sem_s, sem_d, sem_o):
    t = _tile_id()
    sax = lax.axis_index("s")
    base = t * R
    rb14 = (sax * RT) << 14
    zero16f = jnp.zeros((16,), jnp.float32)
    ones16f = jnp.ones((16,), jnp.float32)
    dummy_pk = (lax.iota(jnp.int32, 16) * 64) | (((sax * RT) + R) << 14)

    for i in range(R // 16):
        deg_acc[pl.ds(i * 16, 16)] = zero16f

    pltpu.async_copy(src_hbm.at[pl.ds(0, SCAN)], sbuf.at[pl.ds(0, SCAN)], sem_s)
    pltpu.async_copy(dst_hbm.at[pl.ds(0, SCAN)], dbuf.at[pl.ds(0, SCAN)], sem_d)

    def chunk_body(c, off):
        off = pl.multiple_of(off, 8)
        b = c & 1
        ioff = b * SCAN
        coff = pl.multiple_of(b * CB, 8)

        @pl.when(c + 1 < NCH)
        def _():
            nb = (1 - b) * SCAN
            pltpu.async_copy(src_hbm.at[pl.ds((c + 1) * SCAN, SCAN)],
                             sbuf.at[pl.ds(nb, SCAN)], sem_s)
            pltpu.async_copy(dst_hbm.at[pl.ds((c + 1) * SCAN, SCAN)],
                             dbuf.at[pl.ds(nb, SCAN)], sem_d)

        pltpu.make_async_copy(src_hbm.at[pl.ds(0, SCAN)],
                              sbuf.at[pl.ds(0, SCAN)], sem_s).wait()
        pltpu.make_async_copy(dst_hbm.at[pl.ds(0, SCAN)],
                              dbuf.at[pl.ds(0, SCAN)], sem_d).wait()

        @pl.when(c >= 2)
        def _():
            pltpu.make_async_copy(cbuf.at[pl.ds(0, CB)],
                                  bin_pk.at[pl.ds(0, CB)], sem_o).wait()

        def vreg_body(j, cnt):
            d = dbuf[pl.ds(ioff + j * 16, 16)]
            sv = sbuf[pl.ds(ioff + j * 16, 16)]
            ld = d - base
            m = (ld >= 0) & (ld < R)
            ldc = jnp.where(m, ld, R)
            plsc.addupdate_scatter(deg_acc, [ldc], ones16f, mask=m)
            pk = sv | ((ld << 14) + rb14)
            plsc.store_compressed(cbuf.at[pl.ds(coff + cnt, 16)], pk, mask=m)
            return cnt + plsc.all_reduce_population_count(m)[0]

        cnt = lax.fori_loop(0, SCAN // 16, vreg_body, 0)
        cbuf[pl.ds(coff + cnt, 16)] = dummy_pk
        cnt_pad = (cnt + 7) & ~7
        pltpu.async_copy(cbuf.at[pl.ds(coff, CB)],
                         bin_pk.at[pl.ds(t * CAP + off, CB)], sem_o)
        return off + cnt_pad

    off = pl.multiple_of(lax.fori_loop(0, NCH, chunk_body, 0, unroll=False), 8)
    pltpu.make_async_copy(cbuf.at[pl.ds(0, CB)],
                          bin_pk.at[pl.ds(0, CB)], sem_o).wait()
    pltpu.make_async_copy(cbuf.at[pl.ds(0, CB)],
                          bin_pk.at[pl.ds(0, CB)], sem_o).wait()

    for k in range(IG * GCH // 16):
        cbuf[pl.ds(k * 16, 16)] = dummy_pk
    pltpu.sync_copy(cbuf.at[pl.ds(0, IG * GCH)],
                    bin_pk.at[pl.ds(t * CAP + off, IG * GCH)])
    total = ((off + IG * GCH - 1) // (IG * GCH)) * (IG * GCH)

    cvec[...] = jnp.full((16,), 0, jnp.int32) + total
    pltpu.sync_copy(cvec, counts_hbm.at[pl.ds(t * 16, 16)])
    pltpu.sync_copy(deg_acc, deg_hbm.at[pl.ds(t * R, R)])


def _bin(src, dst):
    f = pl.kernel(
        _bin_body,
        out_type=(
            jax.ShapeDtypeStruct((NT * CAP,), jnp.int32),
            jax.ShapeDtypeStruct((NT * 16,), jnp.int32),
            jax.ShapeDtypeStruct((NPAD,), jnp.float32),
        ),
        mesh=_mesh,
        compiler_params=_sc_params,
        scratch_types=[
            pltpu.VMEM((2 * SCAN,), jnp.int32),
            pltpu.VMEM((2 * SCAN,), jnp.int32),
            pltpu.VMEM((2 * CB,), jnp.int32),
            pltpu.VMEM((R,), jnp.float32),
            pltpu.VMEM((16,), jnp.int32),
            pltpu.SemaphoreType.DMA,
            pltpu.SemaphoreType.DMA,
            pltpu.SemaphoreType.DMA,
        ],
    )
    return f(src, dst)


def _seg_body(d, p_hbm, bin_pk, counts_hbm, acc_hbm,
              pk_stage, src_buf, ldst_buf, rows_v, zbuf, acc_sh, cnt_v,
              sem_i, sem_g, sem_s):
    t = _tile_id()
    sax = lax.axis_index("s")
    rbase = sax * RT
    SG = IG * GCH

    def zrow(r, carry):
        for k in range(d // 16):
            zbuf[r, pl.ds(k * 16, 16)] = jnp.zeros((16,), jnp.float32)
        return carry

    lax.fori_loop(0, RT, zrow, 0, unroll=False)
    pltpu.sync_copy(zbuf, acc_sh.at[pl.ds(rbase, RT)])

    pltpu.sync_copy(counts_hbm.at[pl.ds(t * 16, 16)], cnt_v)
    total = jnp.max(cnt_v[...])
    nch = total // GCH
    tb = pl.multiple_of(t * CAP, 8)

    pltpu.async_copy(bin_pk.at[pl.ds(tb, SG)], pk_stage.at[pl.ds(0, SG)], sem_i)

    def chunk(c, carry):
        b = c & 1
        g = c // IG
        j = c - g * IG
        gslot = g & 1

        @pl.when(j == 0)
        def _():
            pltpu.make_async_copy(bin_pk.at[pl.ds(0, SG)],
                                  pk_stage.at[pl.ds(0, SG)], sem_i).wait()

            @pl.when((g + 1) * SG < total)
            def _():
                hoff = pl.multiple_of(tb + (g + 1) * SG, 8)
                soff2 = pl.multiple_of(((g + 1) & 1) * SG, 8)
                pltpu.async_copy(bin_pk.at[pl.ds(hoff, SG)],
                                 pk_stage.at[pl.ds(soff2, SG)], sem_i)

        @pl.when(c >= 2)
        def _():
            pltpu.make_async_copy(p_hbm.at[pl.ds(0, GCH)], rows_v.at[0],
                                  sem_s).wait()

        soff = gslot * SG + j * GCH
        for k in range(GCH // 16):
            pk = pk_stage[pl.ds(soff + k * 16, 16)]
            src_buf[b, pl.ds(k * 16, 16)] = pk & 16383
            ldst_buf[b, pl.ds(k * 16, 16)] = lax.shift_right_logical(pk, 14)

        pltpu.async_copy(p_hbm.at[src_buf.at[b]], rows_v.at[b], sem_g)

        @pl.when(c >= 1)
        def _():
            pltpu.make_async_copy(p_hbm.at[pl.ds(0, GCH)], rows_v.at[0],
                                  sem_g).wait()
            pltpu.async_copy(rows_v.at[1 - b], acc_sh.at[ldst_buf.at[1 - b]],
                             sem_s, add=True)

        return carry

    lax.fori_loop(0, nch, chunk, 0, unroll=False)

    lastb = (nch - 1) & 1
    pltpu.make_async_copy(p_hbm.at[pl.ds(0, GCH)], rows_v.at[0], sem_g).wait()
    pltpu.async_copy(rows_v.at[lastb], acc_sh.at[ldst_buf.at[lastb]],
                     sem_s, add=True)
    pltpu.make_async_copy(p_hbm.at[pl.ds(0, GCH)], rows_v.at[0], sem_s).wait()
    pltpu.make_async_copy(p_hbm.at[pl.ds(0, GCH)], rows_v.at[0], sem_s).wait()

    pltpu.sync_copy(acc_sh.at[pl.ds(rbase, R)], acc_hbm.at[pl.ds(t * R, R)])


def _seg(p, bin_pk, counts, d):
    f = pl.kernel(
        functools.partial(_seg_body, d),
        out_type=jax.ShapeDtypeStruct((NPAD, d), jnp.float32),
        mesh=_mesh,
        compiler_params=_sc_params,
        scratch_types=[
            pltpu.VMEM((2 * IG * GCH,), jnp.int32),
            pltpu.VMEM((2, GCH), jnp.int32),
            pltpu.VMEM((2, GCH), jnp.int32),
            pltpu.VMEM((2, GCH, d), jnp.float32),
            pltpu.VMEM((RT, d), jnp.float32),
            pltpu.VMEM_SHARED((16 * RT, d), jnp.float32),
            pltpu.VMEM((16,), jnp.int32),
            pltpu.SemaphoreType.DMA,
            pltpu.SemaphoreType.DMA,
            pltpu.SemaphoreType.DMA,
        ],
    )
    return f(p, bin_pk, counts)


BR = 1024


def _ln_rows(h, g, b):
    mu = jnp.mean(h, axis=-1, keepdims=True)
    var = jnp.var(h, axis=-1, keepdims=True)
    return (h - mu) * lax.rsqrt(var + 1e-5) * g + b


_BN_SC = 0.9999950000374997


def _tca_body(x_ref, deg_ref, lng_ref, lnb_ref, w_ref, p_ref):
    dinv = lax.rsqrt(deg_ref[...] + 1.0)
    h = _ln_rows(x_ref[...], lng_ref[...], lnb_ref[...])
    p_ref[...] = (h @ w_ref[...]) * dinv


def _tca(xp, deg, ln_g, ln_b, W1):
    grid = (NPAD // BR,)
    return pl.pallas_call(
        _tca_body,
        grid=grid,
        in_specs=[
            pl.BlockSpec((BR, 128), lambda i: (i, 0)),
            pl.BlockSpec((BR, 1), lambda i: (i, 0)),
            pl.BlockSpec((1, 128), lambda i: (0, 0)),
            pl.BlockSpec((1, 128), lambda i: (0, 0)),
            pl.BlockSpec((128, 128), lambda i: (0, 0)),
        ],
        out_specs=pl.BlockSpec((BR, 128), lambda i: (i, 0)),
        out_shape=jax.ShapeDtypeStruct((NPAD, 128), jnp.float32),
    )(xp, deg, ln_g.reshape(1, -1), ln_b.reshape(1, -1), W1)


def _tcb_body(acc_ref, p_ref, deg_ref, b_ref, g_ref, be_ref, w_ref, out_ref):
    dinv = lax.rsqrt(deg_ref[...] + 1.0)
    pre = (acc_ref[...] + p_ref[...]) * dinv + b_ref[...]
    h = jax.nn.relu(pre * (_BN_SC * g_ref[...]) + be_ref[...])
    out_ref[...] = (h @ w_ref[...]) * dinv


def _tcb(acc, p, deg, b, g, be, W, din, dout):
    grid = (NPAD // BR,)
    return pl.pallas_call(
        _tcb_body,
        grid=grid,
        in_specs=[
            pl.BlockSpec((BR, din), lambda i: (i, 0)),
            pl.BlockSpec((BR, din), lambda i: (i, 0)),
            pl.BlockSpec((BR, 1), lambda i: (i, 0)),
            pl.BlockSpec((1, din), lambda i: (0, 0)),
            pl.BlockSpec((1, din), lambda i: (0, 0)),
            pl.BlockSpec((1, din), lambda i: (0, 0)),
            pl.BlockSpec((din, dout), lambda i: (0, 0)),
        ],
        out_specs=pl.BlockSpec((BR, dout), lambda i: (i, 0)),
        out_shape=jax.ShapeDtypeStruct((NPAD, dout), jnp.float32),
    )(acc, p, deg, b.reshape(1, -1), g.reshape(1, -1), be.reshape(1, -1), W)


def _tce_body(acc_ref, p_ref, deg_ref, b_ref, g_ref, be_ref,
              wc1_ref, bc1_ref, lg1_ref, lb1_ref,
              wc2_ref, bc2_ref, lg2_ref, lb2_ref,
              wc3_ref, bc3_ref, out_ref):
    dinv = lax.rsqrt(deg_ref[...] + 1.0)
    pre = (acc_ref[...] + p_ref[...]) * dinv + b_ref[...]
    h = jax.nn.relu(pre * (_BN_SC * g_ref[...]) + be_ref[...])[:, :32]
    h = jax.nn.relu(_ln_rows(h @ wc1_ref[...] + bc1_ref[...],
                             lg1_ref[...], lb1_ref[...]))
    h = jax.nn.relu(_ln_rows(h @ wc2_ref[...] + bc2_ref[...],
                             lg2_ref[...], lb2_ref[...]))
    out_ref[...] = h @ wc3_ref[...] + bc3_ref[...]


def _tce(acc, p, deg, b4, g4, be4, Wc1, bc1, lg1, lb1, Wc2, bc2, lg2, lb2,
         Wc3, bc3):
    grid = (NPAD // BR,)
    row = lambda v: v.reshape(1, -1)
    full = lambda a, b: pl.BlockSpec((a, b), lambda i: (0, 0))
    return pl.pallas_call(
        _tce_body,
        grid=grid,
        in_specs=[
            pl.BlockSpec((BR, 128), lambda i: (i, 0)),
            pl.BlockSpec((BR, 128), lambda i: (i, 0)),
            pl.BlockSpec((BR, 1), lambda i: (i, 0)),
            full(1, 128), full(1, 128), full(1, 128),
            full(32, 16), full(1, 16), full(1, 16), full(1, 16),
            full(16, 8), full(1, 8), full(1, 8), full(1, 8),
            full(8, 8), full(1, 8),
        ],
        out_specs=pl.BlockSpec((BR, 8), lambda i: (i, 0)),
        out_shape=jax.ShapeDtypeStruct((NPAD, 8), jnp.float32),
    )(acc, p, deg, row(b4), row(g4), row(be4),
      Wc1, row(bc1), row(lg1), row(lb1),
      Wc2, row(bc2), row(lg2), row(lb2),
      Wc3, row(bc3))


def kernel(x, edge_index, ln_g, ln_b, W1, b1, g1, be1, W2, b2, g2, be2, W3, b3,
           g3, be3, W4, b4, g4, be4, Wc1, bc1, lg1, lb1, Wc2, bc2, lg2, lb2,
           Wc3, bc3):
    src = edge_index[0].astype(jnp.int32)
    dst = edge_index[1].astype(jnp.int32)
    bin_pk, counts, degc = _bin(src, dst)
    deg = degc.reshape(NPAD, 1)
    xp = jnp.pad(x, ((0, NPAD - N), (0, 0)))

    W3p = jnp.pad(W3, ((0, 0), (0, 64)))
    b3p = jnp.pad(b3, (0, 64))
    g3p = jnp.pad(g3, (0, 64))
    be3p = jnp.pad(be3, (0, 64))
    W4p = jnp.pad(W4, ((0, 64), (0, 96)))
    b4p = jnp.pad(b4, (0, 96))
    g4p = jnp.pad(g4, (0, 96))
    be4p = jnp.pad(be4, (0, 96))

    p1 = _tca(xp, deg, ln_g, ln_b, W1)
    a1 = _seg(p1, bin_pk, counts, 128)
    p2 = _tcb(a1, p1, deg, b1, g1, be1, W2, 128, 128)
    a2 = _seg(p2, bin_pk, counts, 128)
    p3 = _tcb(a2, p2, deg, b2, g2, be2, W3p, 128, 128)
    a3 = _seg(p3, bin_pk, counts, 128)
    p4 = _tcb(a3, p3, deg, b3p, g3p, be3p, W4p, 128, 128)
    a4 = _seg(p4, bin_pk, counts, 128)
    out = _tce(a4, p4, deg, b4p, g4p, be4p, Wc1, bc1, lg1, lb1,
               Wc2, bc2, lg2, lb2, Wc3, bc3)
    return out[:N]
